# baseline (speedup 1.0000x reference)
import functools

import jax
import jax.numpy as jnp
from jax import lax
from jax.experimental import pallas as pl
from jax.experimental.pallas import tpu as pltpu
from jax.experimental.pallas import tpu_sc as plsc

VOCAB = 1000
VPAD = 1024
DIM = 128
N_IDS = 1024 * 50
NC, NS = 2, 16
NW = NC * NS
B_PER_W = N_IDS // NW
CHUNK = 64
N_CHUNKS = B_PER_W // CHUNK


def _table_body(emb_ref, head_ref, out_ref):
    out_ref[...] = lax.dot_general(
        emb_ref[...], head_ref[...],
        dimension_numbers=(((1,), (1,)), ((), ())),
        preferred_element_type=jnp.float32,
    )


def _make_table(emb, head_padded):
    return pl.pallas_call(
        _table_body,
        out_shape=jax.ShapeDtypeStruct((VOCAB, VPAD), jnp.float32),
    )(emb, head_padded)


def _gather_body(table_hbm, ids_hbm, out_hbm, idx_v, rows_v, sem):
    wid = lax.axis_index("s") * NC + lax.axis_index("c")
    base = wid * B_PER_W
    pltpu.sync_copy(ids_hbm.at[pl.ds(base, B_PER_W)], idx_v)

    def body(c, carry):
        off = pl.multiple_of(c * CHUNK, CHUNK)
        pltpu.async_copy(
            table_hbm.at[idx_v.at[pl.ds(off, CHUNK)]], rows_v, sem).wait()
        pltpu.sync_copy(rows_v, out_hbm.at[pl.ds(base + off, CHUNK)])
        return carry

    lax.fori_loop(0, N_CHUNKS, body, 0)


def _gather_rows(table, flat_ids):
    mesh = plsc.VectorSubcoreMesh(core_axis_name="c", subcore_axis_name="s")
    k = pl.kernel(
        _gather_body,
        out_type=jax.ShapeDtypeStruct((N_IDS, VPAD), jnp.float32),
        mesh=mesh,
        scratch_types=[
            pltpu.VMEM((B_PER_W,), jnp.int32),
            pltpu.VMEM((CHUNK, VPAD), jnp.float32),
            pltpu.SemaphoreType.DMA,
        ],
    )
    return k(table, flat_ids)


def kernel(ids, emb_weight, head_weight):
    b, l = ids.shape
    head_padded = jnp.pad(head_weight, ((0, VPAD - VOCAB), (0, 0)))
    table = _make_table(emb_weight, head_padded)
    flat_ids = ids.reshape(-1).astype(jnp.int32)
    out = _gather_rows(table, flat_ids)
    return out[:, :VOCAB].reshape(b, l, VOCAB)


# speedup vs baseline: 1.0160x; 1.0160x over previous
import functools

import jax
import jax.numpy as jnp
from jax import lax
from jax.experimental import pallas as pl
from jax.experimental.pallas import tpu as pltpu
from jax.experimental.pallas import tpu_sc as plsc

VOCAB = 1000
VPAD = 1024
DIM = 128
N_IDS = 1024 * 50
NC, NS = 2, 16
NW = NC * NS
B_PER_W = N_IDS // NW
CHUNK = 40
N_CHUNKS = B_PER_W // CHUNK


def _table_body(emb_ref, head_ref, out_ref):
    out_ref[...] = lax.dot_general(
        emb_ref[...], head_ref[...],
        dimension_numbers=(((1,), (1,)), ((), ())),
        preferred_element_type=jnp.float32,
    )


def _make_table(emb, head_padded):
    return pl.pallas_call(
        _table_body,
        out_shape=jax.ShapeDtypeStruct((VOCAB, VPAD), jnp.float32),
    )(emb, head_padded)


def _gather_body(table_hbm, ids_hbm, out_hbm, idx_v, rows0, rows1, tail_v,
                 sem0, sem1):
    wid = lax.axis_index("s") * NC + lax.axis_index("c")
    base = wid * B_PER_W
    pltpu.sync_copy(ids_hbm.at[pl.ds(base, B_PER_W)], idx_v)

    rows = (rows0, rows1)
    sems = (sem0, sem1)

    def _start(c, buf):
        off = pl.multiple_of(c * CHUNK, CHUNK)
        pltpu.async_copy(
            table_hbm.at[idx_v.at[pl.ds(off, CHUNK)]], rows[buf], sems[buf])

    def _drain_and_store(c, buf, tail_v):
        off = pl.multiple_of(c * CHUNK, CHUNK)
        pltpu.make_async_copy(
            table_hbm.at[idx_v.at[pl.ds(off, CHUNK)]],
            rows[buf], sems[buf]).wait()
        def fill_tail(r, carry):
            for j in range(6):
                tail_v[r, pl.ds(j * 16, 16)] = \
                    rows[buf][r, pl.ds(896 + j * 16, 16)]
            tail_v[r, pl.ds(88, 16)] = rows[buf][r, pl.ds(984, 16)]
            return carry

        lax.fori_loop(0, CHUNK, fill_tail, 0)
        pltpu.sync_copy(rows[buf].at[:, pl.ds(0, 896)],
                        out_hbm.at[pl.ds(base + off, CHUNK), pl.ds(0, 896)])
        pltpu.sync_copy(tail_v,
                        out_hbm.at[pl.ds(base + off, CHUNK), pl.ds(896, 104)])

    _start(0, 0)

    def body(c, carry):
        @pl.when(c + 1 < N_CHUNKS)
        def _():
            @pl.when(c % 2 == 0)
            def _():
                _start(c + 1, 1)

            @pl.when(c % 2 == 1)
            def _():
                _start(c + 1, 0)

        @pl.when(c % 2 == 0)
        def _():
            _drain_and_store(c, 0, tail_v)

        @pl.when(c % 2 == 1)
        def _():
            _drain_and_store(c, 1, tail_v)

        return carry

    lax.fori_loop(0, N_CHUNKS, body, 0)


def _gather_rows(table, flat_ids):
    mesh = plsc.VectorSubcoreMesh(core_axis_name="c", subcore_axis_name="s")
    k = pl.kernel(
        _gather_body,
        out_type=jax.ShapeDtypeStruct((N_IDS, VOCAB), jnp.float32),
        mesh=mesh,
        scratch_types=[
            pltpu.VMEM((B_PER_W,), jnp.int32),
            pltpu.VMEM((CHUNK, VPAD), jnp.float32),
            pltpu.VMEM((CHUNK, VPAD), jnp.float32),
            pltpu.VMEM((CHUNK, 104), jnp.float32),
            pltpu.SemaphoreType.DMA,
            pltpu.SemaphoreType.DMA,
        ],
    )
    return k(table, flat_ids)


def kernel(ids, emb_weight, head_weight):
    b, l = ids.shape
    head_padded = jnp.pad(head_weight, ((0, VPAD - VOCAB), (0, 0)))
    table = _make_table(emb_weight, head_padded)
    flat_ids = ids.reshape(-1).astype(jnp.int32)
    out = _gather_rows(table, flat_ids)
    return out.reshape(b, l, VOCAB)


# speedup vs baseline: 1.2460x; 1.2264x over previous
import functools

import jax
import jax.numpy as jnp
from jax import lax
from jax.experimental import pallas as pl
from jax.experimental.pallas import tpu as pltpu
from jax.experimental.pallas import tpu_sc as plsc

VOCAB = 1000
VPAD = 1024
ALIGNED = 896
TAIL = VOCAB - ALIGNED
DIM = 128
BATCH = 1024
HIST = 50
HIST_PAD = 56
NC, NS = 2, 16
NW = NC * NS
E_PER_W = BATCH // NW


def _table_body(emb_ref, head_ref, out_ref):
    out_ref[...] = lax.dot_general(
        emb_ref[...], head_ref[...],
        dimension_numbers=(((1,), (1,)), ((), ())),
        preferred_element_type=jnp.float32,
    )


def _make_table(emb, head_padded):
    return pl.pallas_call(
        _table_body,
        out_shape=jax.ShapeDtypeStruct((VOCAB, VPAD), jnp.float32),
    )(emb, head_padded)


def _gather_body(table_hbm, ids_hbm, out_hbm, idx_v, rows0, rows1, tail_v,
                 sem0, sem1):
    wid = lax.axis_index("s") * NC + lax.axis_index("c")
    base = wid * E_PER_W
    pltpu.sync_copy(ids_hbm.at[pl.ds(base * HIST_PAD, E_PER_W * HIST_PAD)],
                    idx_v)

    rows = (rows0, rows1)
    sems = (sem0, sem1)

    def _start(c, buf):
        off = pl.multiple_of(c * HIST_PAD, 8)
        pltpu.async_copy(
            table_hbm.at[idx_v.at[pl.ds(off, HIST)]], rows[buf], sems[buf])

    def _drain_and_store(c, buf):
        e = base + c
        pltpu.make_async_copy(
            table_hbm.at[idx_v.at[pl.ds(pl.multiple_of(c * HIST_PAD, 8),
                                        HIST)]],
            rows[buf], sems[buf]).wait()

        def fill_tail(r, carry):
            for j in range(6):
                tail_v[r, pl.ds(j * 16, 16)] = \
                    rows[buf][r, pl.ds(ALIGNED + j * 16, 16)]
            tail_v[r, pl.ds(TAIL - 16, 16)] = \
                rows[buf][r, pl.ds(VOCAB - 16, 16)]
            return carry

        lax.fori_loop(0, HIST, fill_tail, 0)
        pltpu.sync_copy(rows[buf].at[:, pl.ds(0, ALIGNED)],
                        out_hbm.at[e, :, pl.ds(0, ALIGNED)])
        pltpu.sync_copy(tail_v, out_hbm.at[e, :, pl.ds(ALIGNED, TAIL)])

    _start(0, 0)

    def body(c, carry):
        @pl.when(c + 1 < E_PER_W)
        def _():
            @pl.when(c % 2 == 0)
            def _():
                _start(c + 1, 1)

            @pl.when(c % 2 == 1)
            def _():
                _start(c + 1, 0)

        @pl.when(c % 2 == 0)
        def _():
            _drain_and_store(c, 0)

        @pl.when(c % 2 == 1)
        def _():
            _drain_and_store(c, 1)

        return carry

    lax.fori_loop(0, E_PER_W, body, 0)


def _gather_rows(table, flat_ids):
    mesh = plsc.VectorSubcoreMesh(core_axis_name="c", subcore_axis_name="s")
    k = pl.kernel(
        _gather_body,
        out_type=jax.ShapeDtypeStruct((BATCH, HIST, VOCAB), jnp.float32),
        mesh=mesh,
        scratch_types=[
            pltpu.VMEM((E_PER_W * HIST_PAD,), jnp.int32),
            pltpu.VMEM((HIST, VPAD), jnp.float32),
            pltpu.VMEM((HIST, VPAD), jnp.float32),
            pltpu.VMEM((HIST, TAIL), jnp.float32),
            pltpu.SemaphoreType.DMA,
            pltpu.SemaphoreType.DMA,
        ],
    )
    return k(table, flat_ids)


def kernel(ids, emb_weight, head_weight):
    head_padded = jnp.pad(head_weight, ((0, VPAD - VOCAB), (0, 0)))
    table = _make_table(emb_weight, head_padded)
    ids_padded = jnp.pad(ids.astype(jnp.int32),
                         ((0, 0), (0, HIST_PAD - HIST)))
    return _gather_rows(table, ids_padded.reshape(-1))
